# baseline (speedup 1.0000x reference)
import jax
import jax.numpy as jnp
from jax.experimental import pallas as pl
from jax.experimental.pallas import tpu as pltpu

N = 8192
D = 64
BM = 512


def _pool_kernel(adj_ref, xb_ref, xr_ref, o_ref):
    a = adj_ref[...].astype(jnp.bfloat16)
    acc = jnp.dot(a, xb_ref[...], preferred_element_type=jnp.float32)
    o_ref[...] = xr_ref[...] + acc


def kernel(x, adj):
    xb = x.astype(jnp.bfloat16)
    return pl.pallas_call(
        _pool_kernel,
        grid=(N // BM,),
        in_specs=[
            pl.BlockSpec((BM, N), lambda i: (i, 0)),
            pl.BlockSpec((N, D), lambda i: (0, 0)),
            pl.BlockSpec((BM, D), lambda i: (i, 0)),
        ],
        out_specs=pl.BlockSpec((BM, D), lambda i: (i, 0)),
        out_shape=jax.ShapeDtypeStruct((N, D), jnp.float32),
        compiler_params=pltpu.CompilerParams(
            dimension_semantics=("parallel",),
        ),
    )(adj, xb, x)


# speedup vs baseline: 1.0615x; 1.0615x over previous
import jax
import jax.numpy as jnp
from jax.experimental import pallas as pl
from jax.experimental.pallas import tpu as pltpu

N = 8192
D = 64
BM = 256
NH = N // 2


def _pool_kernel(adj0_ref, adj1_ref, xb_ref, xr_ref, o_ref):
    a0 = adj0_ref[...].astype(jnp.bfloat16)
    a1 = adj1_ref[...].astype(jnp.bfloat16)
    acc = jnp.dot(a0, xb_ref[:NH, :], preferred_element_type=jnp.float32)
    acc += jnp.dot(a1, xb_ref[NH:, :], preferred_element_type=jnp.float32)
    o_ref[...] = xr_ref[...] + acc


def kernel(x, adj):
    xb = x.astype(jnp.bfloat16)
    return pl.pallas_call(
        _pool_kernel,
        grid=(N // BM,),
        in_specs=[
            pl.BlockSpec((BM, NH), lambda i: (i, 0)),
            pl.BlockSpec((BM, NH), lambda i: (i, 1)),
            pl.BlockSpec((N, D), lambda i: (0, 0)),
            pl.BlockSpec((BM, D), lambda i: (i, 0)),
        ],
        out_specs=pl.BlockSpec((BM, D), lambda i: (i, 0)),
        out_shape=jax.ShapeDtypeStruct((N, D), jnp.float32),
        compiler_params=pltpu.CompilerParams(
            dimension_semantics=("arbitrary",),
        ),
    )(adj, adj, xb, x)
